# baseline (speedup 1.0000x reference)
import functools
import numpy as np
import jax
import jax.numpy as jnp
from jax import lax
from jax.experimental import pallas as pl
from jax.experimental.pallas import tpu as pltpu
from jax.experimental.pallas import tpu_sc as plsc

B, L = 4096, 200
POI = 100001
EMBED = 128
EHALF = EMBED // 2
HOUR_DIM = 32
HSTRIDE = 32
_RB = 1024
PAD_POI = 100352
_NSTEP = PAD_POI // _RB

NW = 32
TOTAL = B * L
PER_W = TOTAL // NW
C = 128
NCHUNK = PER_W // C


def _sinusoidal_pe(seq_len, d_model):
    pos = np.arange(seq_len, dtype=np.float32)[:, None]
    div = np.exp(np.arange(0, d_model, 2, dtype=np.float32) * (-np.log(10000.0) / d_model))
    pe = np.zeros((seq_len, d_model), dtype=np.float32)
    pe[:, 0::2] = np.sin(pos * div)
    pe[:, 1::2] = np.cos(pos * div)
    return pe


_PERM = np.concatenate(
    [np.arange(32 * t, 32 * t + 16) for t in range(4)]
    + [np.arange(32 * t + 16, 32 * t + 32) for t in range(4)]
)
_PE = _sinusoidal_pe(L, EMBED)[:, _PERM]


def _pack_u32(y):
    yb = y.astype(jnp.bfloat16)
    lo = lax.bitcast_convert_type(yb[:, :EHALF], jnp.uint16).astype(jnp.uint32)
    hi = lax.bitcast_convert_type(yb[:, EHALF:], jnp.uint16).astype(jnp.uint32)
    return lo | (hi << 16)


def _precompute_body(pt_ref, w1_ref, ht_ref, w2_ref, b_ref, pe_ref, poi_out, hp_out):
    i = pl.program_id(0)
    x = pt_ref[...]
    rid = lax.broadcasted_iota(jnp.int32, (_RB, 1), 0) + i * _RB
    x = jnp.where(rid == 0, 0.0, x)
    poi_out[...] = _pack_u32(
        jnp.dot(x, w1_ref[...], preferred_element_type=jnp.float32))

    @pl.when(i == 0)
    def _():
        h = ht_ref[...]
        hid = lax.broadcasted_iota(jnp.int32, (32, 1), 0)
        h = jnp.where(hid == 0, 0.0, h)
        hc = jnp.dot(h, w2_ref[...], preferred_element_type=jnp.float32) + b_ref[...]
        hp = pe_ref[...][:, None, :] + hc[None, :, :]
        hp_out[...] = _pack_u32(hp.reshape(L * HSTRIDE, EMBED))


_precompute = pl.pallas_call(
    _precompute_body,
    grid=(_NSTEP,),
    in_specs=[
        pl.BlockSpec((_RB, EMBED), lambda i: (i, 0)),
        pl.BlockSpec((EMBED, EMBED), lambda i: (0, 0)),
        pl.BlockSpec((32, EMBED), lambda i: (0, 0)),
        pl.BlockSpec((EMBED, EMBED), lambda i: (0, 0)),
        pl.BlockSpec((1, EMBED), lambda i: (0, 0)),
        pl.BlockSpec((L, EMBED), lambda i: (0, 0)),
    ],
    out_specs=[
        pl.BlockSpec((_RB, EHALF), lambda i: (i, 0)),
        pl.BlockSpec((L * HSTRIDE, EHALF), lambda i: (0, 0)),
    ],
    out_shape=[
        jax.ShapeDtypeStruct((PAD_POI, EHALF), jnp.uint32),
        jax.ShapeDtypeStruct((L * HSTRIDE, EHALF), jnp.uint32),
    ],
)


def _sc_body(tok_hbm, hour_hbm, poi_hbm, hp_hbm, out_hbm,
             tok_v, idx2_v, a0, a1, b0, b1, o0, o1,
             sa0, sa1, sb0, sb1, so0, so1):
    cid = lax.axis_index("c")
    sid = lax.axis_index("s")
    wid = sid * 2 + cid
    base = wid * PER_W

    abuf = (a0, a1)
    bbuf = (b0, b1)
    obuf = (o0, o1)
    sa = (sa0, sa1)
    sb = (sb0, sb1)
    so = (so0, so1)

    pltpu.sync_copy(tok_hbm.at[pl.ds(base, PER_W)], tok_v)
    pltpu.sync_copy(hour_hbm.at[pl.ds(base, PER_W)], idx2_v)

    l0 = lax.rem(base + lax.iota(jnp.int32, 16), L)

    def mk_idx(k, l):
        off = k * 16
        idx2_v[pl.ds(off, 16)] = l * HSTRIDE + idx2_v[pl.ds(off, 16)]
        ln = l + 16
        return jnp.where(ln >= L, ln - L, ln)

    lax.fori_loop(0, PER_W // 16, mk_idx, l0)

    def issue(g, j):
        pltpu.async_copy(poi_hbm.at[tok_v.at[pl.ds(g * C, C)]], abuf[j], sa[j])
        pltpu.async_copy(hp_hbm.at[idx2_v.at[pl.ds(g * C, C)]], bbuf[j], sb[j])

    def wait_gathers(j):
        pltpu.make_async_copy(poi_hbm.at[tok_v.at[pl.ds(0, C)]], abuf[j],
                              sa[j]).wait()
        pltpu.make_async_copy(hp_hbm.at[idx2_v.at[pl.ds(0, C)]], bbuf[j],
                              sb[j]).wait()

    def issue_out(g, j):
        pltpu.async_copy(obuf[j], out_hbm.at[pl.ds(base + g * C, C)], so[j])

    def wait_out(j):
        pltpu.make_async_copy(obuf[j], out_hbm.at[pl.ds(base, C)], so[j]).wait()

    c0 = jnp.float32(0.9870664279131003)
    c1 = jnp.float32(-0.27682685756720027)
    c2 = jnp.float32(0.05897604278788518)
    c3 = jnp.float32(-0.005353628855466359)
    hmask = jnp.uint32(0xFFFF0000)

    def tanh_poly(x):
        x = jnp.minimum(jnp.maximum(x, -2.0), 2.0)
        u = x * x
        return x * (((c3 * u + c2) * u + c1) * u + c0)

    def compute(j):
        aa, bb, ov = abuf[j], bbuf[j], obuf[j]

        def row(r, _):
            for k in range(EMBED // 32):
                wa = aa[r, pl.ds(k * 16, 16)]
                wb = bb[r, pl.ds(k * 16, 16)]
                bc = lambda v: lax.bitcast_convert_type(v, jnp.float32)
                xlo = bc(wa << 16) + bc(wb << 16)
                xhi = bc(wa & hmask) + bc(wb & hmask)
                ov[r, pl.ds(k * 32, 16)] = tanh_poly(xlo)
                ov[r, pl.ds(k * 32 + 16, 16)] = tanh_poly(xhi)
            return 0

        lax.fori_loop(0, C, row, 0)

    NH = NCHUNK // 2

    issue(0, 0)

    def pair(i, _):
        for b in range(2):
            g2 = i * 2 + b
            p, q = b, 1 - b
            if b == 0:
                issue(g2 + 1, q)
            else:

                @pl.when(i < NH - 1)
                def _(g2=g2, q=q):
                    issue(g2 + 1, q)
            wait_gathers(p)

            @pl.when(i >= 1)
            def _(p=p):
                wait_out(p)

            compute(p)
            issue_out(g2, p)
        return 0

    lax.fori_loop(0, NH, pair, 0)
    wait_out(0)
    wait_out(1)


_sc_gather = functools.partial(
    pl.kernel,
    out_type=jax.ShapeDtypeStruct((TOTAL, EMBED), jnp.float32),
    mesh=plsc.VectorSubcoreMesh(core_axis_name="c", subcore_axis_name="s"),
    compiler_params=pltpu.CompilerParams(use_tc_tiling_on_sc=False),
    scratch_types=[
        pltpu.VMEM((PER_W,), jnp.int32),
        pltpu.VMEM((PER_W,), jnp.int32),
        pltpu.VMEM((C, EHALF), jnp.uint32),
        pltpu.VMEM((C, EHALF), jnp.uint32),
        pltpu.VMEM((C, EHALF), jnp.uint32),
        pltpu.VMEM((C, EHALF), jnp.uint32),
        pltpu.VMEM((C, EMBED), jnp.float32),
        pltpu.VMEM((C, EMBED), jnp.float32),
        pltpu.SemaphoreType.DMA,
        pltpu.SemaphoreType.DMA,
        pltpu.SemaphoreType.DMA,
        pltpu.SemaphoreType.DMA,
        pltpu.SemaphoreType.DMA,
        pltpu.SemaphoreType.DMA,
    ],
)(_sc_body)


@jax.jit
def kernel(toeken_seq, hour_seq, poi_table, hour_table, fc_W, fc_b):
    tok = toeken_seq.reshape(-1).astype(jnp.int32)
    hour = hour_seq.reshape(-1).astype(jnp.int32)
    pt_pad = jnp.pad(poi_table, ((0, PAD_POI - POI), (0, 0)))
    ht_pad = jnp.zeros((32, EMBED), jnp.float32).at[:25, :HOUR_DIM].set(hour_table)
    w1 = fc_W[:EMBED, _PERM]
    w2_pad = jnp.zeros((EMBED, EMBED), jnp.float32).at[:HOUR_DIM].set(fc_W[EMBED:])
    w2_pad = w2_pad[:, _PERM]
    poi_c, hp = _precompute(pt_pad, w1, ht_pad, w2_pad,
                            fc_b[_PERM].reshape(1, EMBED), _PE)
    out = _sc_gather(tok, hour, poi_c, hp)
    return out.reshape(B, L, EMBED)


# speedup vs baseline: 3.7246x; 3.7246x over previous
import functools
import numpy as np
import jax
import jax.numpy as jnp
from jax import lax
from jax.experimental import pallas as pl
from jax.experimental.pallas import tpu as pltpu
from jax.experimental.pallas import tpu_sc as plsc

B, L = 4096, 200
POI = 100001
EMBED = 128
HOUR_DIM = 32
HSTRIDE = 32
_RB = 1024
PAD_POI = 100352
_NSTEP = PAD_POI // _RB

NW = 32
TOTAL = B * L
PER_W = TOTAL // NW
C = 128
NCHUNK = PER_W // C


def _sinusoidal_pe(seq_len, d_model):
    pos = np.arange(seq_len, dtype=np.float32)[:, None]
    div = np.exp(np.arange(0, d_model, 2, dtype=np.float32) * (-np.log(10000.0) / d_model))
    pe = np.zeros((seq_len, d_model), dtype=np.float32)
    pe[:, 0::2] = np.sin(pos * div)
    pe[:, 1::2] = np.cos(pos * div)
    return pe


_PE = _sinusoidal_pe(L, EMBED)


def _precompute_body(pt_ref, w1_ref, ht_ref, w2_ref, b_ref, pe_ref, poi_out, hp_out):
    i = pl.program_id(0)
    x = pt_ref[...]
    rid = lax.broadcasted_iota(jnp.int32, (_RB, 1), 0) + i * _RB
    x = jnp.where(rid == 0, 0.0, x)
    poi_out[...] = jnp.dot(x, w1_ref[...], preferred_element_type=jnp.float32)

    @pl.when(i == 0)
    def _():
        h = ht_ref[...]
        hid = lax.broadcasted_iota(jnp.int32, (32, 1), 0)
        h = jnp.where(hid == 0, 0.0, h)
        hc = jnp.dot(h, w2_ref[...], preferred_element_type=jnp.float32) + b_ref[...]
        hp = pe_ref[...][:, None, :] + hc[None, :, :]
        hp_out[...] = hp.reshape(L * HSTRIDE, EMBED)


_precompute = pl.pallas_call(
    _precompute_body,
    grid=(_NSTEP,),
    in_specs=[
        pl.BlockSpec((_RB, EMBED), lambda i: (i, 0)),
        pl.BlockSpec((EMBED, EMBED), lambda i: (0, 0)),
        pl.BlockSpec((32, EMBED), lambda i: (0, 0)),
        pl.BlockSpec((EMBED, EMBED), lambda i: (0, 0)),
        pl.BlockSpec((1, EMBED), lambda i: (0, 0)),
        pl.BlockSpec((L, EMBED), lambda i: (0, 0)),
    ],
    out_specs=[
        pl.BlockSpec((_RB, EMBED), lambda i: (i, 0)),
        pl.BlockSpec((L * HSTRIDE, EMBED), lambda i: (0, 0)),
    ],
    out_shape=[
        jax.ShapeDtypeStruct((PAD_POI, EMBED), jnp.float32),
        jax.ShapeDtypeStruct((L * HSTRIDE, EMBED), jnp.float32),
    ],
)


_NBUF = 4


def _sc_body(tok_hbm, hour_hbm, poi_hbm, hp_hbm, out_hbm,
             tok_v, idx2_v, rows0, rows1, rows2, rows3,
             sg0, sg1, sg2, sg3, so0, so1, so2, so3):
    cid = lax.axis_index("c")
    sid = lax.axis_index("s")
    wid = sid * 2 + cid
    base = wid * PER_W

    rows = (rows0, rows1, rows2, rows3)
    sg = (sg0, sg1, sg2, sg3)
    so = (so0, so1, so2, so3)

    pltpu.sync_copy(tok_hbm.at[pl.ds(base, PER_W)], tok_v)
    pltpu.sync_copy(hour_hbm.at[pl.ds(base, PER_W)], idx2_v)

    l0 = lax.rem(base + lax.iota(jnp.int32, 16), L)

    def mk_idx(k, l):
        for t in range(2):
            off = k * 32 + t * 16
            idx2_v[pl.ds(off, 16)] = l * HSTRIDE + idx2_v[pl.ds(off, 16)]
            ln = l + 16
            l = jnp.where(ln >= L, ln - L, ln)
        return l

    lax.fori_loop(0, PER_W // 32, mk_idx, l0)

    def issue_a(g, j):
        pltpu.async_copy(poi_hbm.at[tok_v.at[pl.ds(g * C, C)]], rows[j], sg[j])

    def issue_b(g, j):
        pltpu.async_copy(hp_hbm.at[idx2_v.at[pl.ds(g * C, C)]], rows[j], sg[j],
                         add=True)

    def wait_g(j):
        pltpu.make_async_copy(poi_hbm.at[tok_v.at[pl.ds(0, C)]], rows[j],
                              sg[j]).wait()

    def issue_out(g, j):
        pltpu.async_copy(rows[j], out_hbm.at[pl.ds(base + g * C, C)], so[j])

    def wait_out(j):
        pltpu.make_async_copy(rows[j], out_hbm.at[pl.ds(base, C)], so[j]).wait()

    c0 = jnp.float32(0.9963463153606634)
    c1 = jnp.float32(-0.3105520803637966)
    c2 = jnp.float32(0.09100609831812505)
    c3 = jnp.float32(-0.016430265990737714)
    c4 = jnp.float32(0.0012641228580639412)

    def compute(j):
        ra = rows[j]

        @plsc.parallel_loop(0, C, unroll=2)
        def _row(r):
            for k in range(EMBED // 16):
                sl = pl.ds(k * 16, 16)
                x = ra[r, sl]
                x = jnp.minimum(jnp.maximum(x, -2.0), 2.0)
                u = x * x
                p = (((c4 * u + c3) * u + c2) * u + c1) * u + c0
                ra[r, sl] = x * p

    NQ = NCHUNK // _NBUF

    issue_a(0, 0)
    issue_a(1, 1)
    wait_g(0)
    issue_b(0, 0)

    def quad(i, _):
        for b in range(_NBUF):
            ja, jb, jc = (b + 2) % _NBUF, (b + 1) % _NBUF, b
            if b >= 2:
                wait_out(ja)

                @pl.when(i < NQ - 1)
                def _(b=b, ja=ja, i_=i):
                    issue_a(i_ * _NBUF + b + 2, ja)
            else:

                @pl.when(i >= 1)
                def _(ja=ja):
                    wait_out(ja)

                issue_a(i * _NBUF + b + 2, ja)
            if b < 3:
                wait_g(jb)
                issue_b(i * _NBUF + b + 1, jb)
            else:

                @pl.when(i < NQ - 1)
                def _(jb=jb, i_=i):
                    wait_g(jb)
                    issue_b(i_ * _NBUF + b + 1, jb)
            wait_g(jc)
            compute(jc)
            issue_out(i * _NBUF + b, jc)
        return 0

    lax.fori_loop(0, NQ, quad, 0)
    wait_out(2)
    wait_out(3)


_sc_gather = functools.partial(
    pl.kernel,
    out_type=jax.ShapeDtypeStruct((TOTAL, EMBED), jnp.float32),
    mesh=plsc.VectorSubcoreMesh(core_axis_name="c", subcore_axis_name="s"),
    scratch_types=[
        pltpu.VMEM((PER_W,), jnp.int32),
        pltpu.VMEM((PER_W,), jnp.int32),
        pltpu.VMEM((C, EMBED), jnp.float32),
        pltpu.VMEM((C, EMBED), jnp.float32),
        pltpu.VMEM((C, EMBED), jnp.float32),
        pltpu.VMEM((C, EMBED), jnp.float32),
        pltpu.SemaphoreType.DMA,
        pltpu.SemaphoreType.DMA,
        pltpu.SemaphoreType.DMA,
        pltpu.SemaphoreType.DMA,
        pltpu.SemaphoreType.DMA,
        pltpu.SemaphoreType.DMA,
        pltpu.SemaphoreType.DMA,
        pltpu.SemaphoreType.DMA,
    ],
)(_sc_body)


@jax.jit
def kernel(toeken_seq, hour_seq, poi_table, hour_table, fc_W, fc_b):
    tok = toeken_seq.reshape(-1).astype(jnp.int32)
    hour = hour_seq.reshape(-1).astype(jnp.int32)
    ht_pad = jnp.zeros((32, EMBED), jnp.float32).at[:25, :HOUR_DIM].set(hour_table)
    w1 = fc_W[:EMBED]
    w2_pad = jnp.zeros((EMBED, EMBED), jnp.float32).at[:HOUR_DIM].set(fc_W[EMBED:])
    poi_c, hp = _precompute(poi_table, w1, ht_pad, w2_pad,
                            fc_b.reshape(1, EMBED), _PE)
    out = _sc_gather(tok, hour, poi_c, hp)
    return out.reshape(B, L, EMBED)


# speedup vs baseline: 3.7258x; 1.0003x over previous
import functools
import numpy as np
import jax
import jax.numpy as jnp
from jax import lax
from jax.experimental import pallas as pl
from jax.experimental.pallas import tpu as pltpu
from jax.experimental.pallas import tpu_sc as plsc

B, L = 4096, 200
POI = 100001
EMBED = 128
HOUR_DIM = 32
HSTRIDE = 32
_RB = 1024
PAD_POI = 100352
_NSTEP = PAD_POI // _RB

NW = 32
TOTAL = B * L
PER_W = TOTAL // NW
C = 128
NCHUNK = PER_W // C


def _sinusoidal_pe(seq_len, d_model):
    pos = np.arange(seq_len, dtype=np.float32)[:, None]
    div = np.exp(np.arange(0, d_model, 2, dtype=np.float32) * (-np.log(10000.0) / d_model))
    pe = np.zeros((seq_len, d_model), dtype=np.float32)
    pe[:, 0::2] = np.sin(pos * div)
    pe[:, 1::2] = np.cos(pos * div)
    return pe


_PE = _sinusoidal_pe(L, EMBED)


def _precompute_body(pt_ref, w1_ref, ht_ref, w2_ref, b_ref, pe_ref, poi_out, hp_out):
    i = pl.program_id(0)
    x = pt_ref[...]
    rid = lax.broadcasted_iota(jnp.int32, (_RB, 1), 0) + i * _RB
    x = jnp.where(rid == 0, 0.0, x)
    poi_out[...] = jnp.dot(x.astype(jnp.bfloat16),
                           w1_ref[...].astype(jnp.bfloat16),
                           preferred_element_type=jnp.float32)

    @pl.when(i == 0)
    def _():
        h = ht_ref[...]
        hid = lax.broadcasted_iota(jnp.int32, (32, 1), 0)
        h = jnp.where(hid == 0, 0.0, h)
        hc = jnp.dot(h, w2_ref[...], preferred_element_type=jnp.float32) + b_ref[...]
        hp = pe_ref[...][:, None, :] + hc[None, :, :]
        hp_out[...] = hp.reshape(L * HSTRIDE, EMBED)


_precompute = pl.pallas_call(
    _precompute_body,
    grid=(_NSTEP,),
    in_specs=[
        pl.BlockSpec((_RB, EMBED), lambda i: (i, 0)),
        pl.BlockSpec((EMBED, EMBED), lambda i: (0, 0)),
        pl.BlockSpec((32, EMBED), lambda i: (0, 0)),
        pl.BlockSpec((EMBED, EMBED), lambda i: (0, 0)),
        pl.BlockSpec((1, EMBED), lambda i: (0, 0)),
        pl.BlockSpec((L, EMBED), lambda i: (0, 0)),
    ],
    out_specs=[
        pl.BlockSpec((_RB, EMBED), lambda i: (i, 0)),
        pl.BlockSpec((L * HSTRIDE, EMBED), lambda i: (0, 0)),
    ],
    out_shape=[
        jax.ShapeDtypeStruct((PAD_POI, EMBED), jnp.float32),
        jax.ShapeDtypeStruct((L * HSTRIDE, EMBED), jnp.float32),
    ],
)


_NBUF = 4


def _sc_body(tok_hbm, hour_hbm, poi_hbm, hp_hbm, out_hbm,
             tok_v, idx2_v, rows0, rows1, rows2, rows3,
             sg0, sg1, sg2, sg3, so0, so1, so2, so3):
    cid = lax.axis_index("c")
    sid = lax.axis_index("s")
    wid = sid * 2 + cid
    base = wid * PER_W

    rows = (rows0, rows1, rows2, rows3)
    sg = (sg0, sg1, sg2, sg3)
    so = (so0, so1, so2, so3)

    pltpu.sync_copy(tok_hbm.at[pl.ds(base, PER_W)], tok_v)
    pltpu.sync_copy(hour_hbm.at[pl.ds(base, PER_W)], idx2_v)

    l0 = lax.rem(base + lax.iota(jnp.int32, 16), L)

    def mk_idx(k, l):
        for t in range(2):
            off = k * 32 + t * 16
            idx2_v[pl.ds(off, 16)] = l * HSTRIDE + idx2_v[pl.ds(off, 16)]
            ln = l + 16
            l = jnp.where(ln >= L, ln - L, ln)
        return l

    lax.fori_loop(0, PER_W // 32, mk_idx, l0)

    def issue_a(g, j):
        pltpu.async_copy(poi_hbm.at[tok_v.at[pl.ds(g * C, C)]], rows[j], sg[j])

    def issue_b(g, j):
        pltpu.async_copy(hp_hbm.at[idx2_v.at[pl.ds(g * C, C)]], rows[j], sg[j],
                         add=True)

    def wait_g(j):
        pltpu.make_async_copy(poi_hbm.at[tok_v.at[pl.ds(0, C)]], rows[j],
                              sg[j]).wait()

    def issue_out(g, j):
        pltpu.async_copy(rows[j], out_hbm.at[pl.ds(base + g * C, C)], so[j])

    def wait_out(j):
        pltpu.make_async_copy(rows[j], out_hbm.at[pl.ds(base, C)], so[j]).wait()

    c0 = jnp.float32(0.9963463153606634)
    c1 = jnp.float32(-0.3105520803637966)
    c2 = jnp.float32(0.09100609831812505)
    c3 = jnp.float32(-0.016430265990737714)
    c4 = jnp.float32(0.0012641228580639412)

    def compute(j):
        ra = rows[j]

        @plsc.parallel_loop(0, C, unroll=2)
        def _row(r):
            for k in range(EMBED // 16):
                sl = pl.ds(k * 16, 16)
                x = ra[r, sl]
                x = jnp.minimum(jnp.maximum(x, -2.0), 2.0)
                u = x * x
                p = (((c4 * u + c3) * u + c2) * u + c1) * u + c0
                ra[r, sl] = x * p

    NQ = NCHUNK // _NBUF

    issue_a(0, 0)
    issue_a(1, 1)
    wait_g(0)
    issue_b(0, 0)

    def quad(i, _):
        for b in range(_NBUF):
            ja, jb, jc = (b + 2) % _NBUF, (b + 1) % _NBUF, b
            if b >= 2:
                wait_out(ja)

                @pl.when(i < NQ - 1)
                def _(b=b, ja=ja, i_=i):
                    issue_a(i_ * _NBUF + b + 2, ja)
            else:

                @pl.when(i >= 1)
                def _(ja=ja):
                    wait_out(ja)

                issue_a(i * _NBUF + b + 2, ja)
            if b < 3:
                wait_g(jb)
                issue_b(i * _NBUF + b + 1, jb)
            else:

                @pl.when(i < NQ - 1)
                def _(jb=jb, i_=i):
                    wait_g(jb)
                    issue_b(i_ * _NBUF + b + 1, jb)
            wait_g(jc)
            compute(jc)
            issue_out(i * _NBUF + b, jc)
        return 0

    lax.fori_loop(0, NQ, quad, 0)
    wait_out(2)
    wait_out(3)


_sc_gather = functools.partial(
    pl.kernel,
    out_type=jax.ShapeDtypeStruct((TOTAL, EMBED), jnp.float32),
    mesh=plsc.VectorSubcoreMesh(core_axis_name="c", subcore_axis_name="s"),
    scratch_types=[
        pltpu.VMEM((PER_W,), jnp.int32),
        pltpu.VMEM((PER_W,), jnp.int32),
        pltpu.VMEM((C, EMBED), jnp.float32),
        pltpu.VMEM((C, EMBED), jnp.float32),
        pltpu.VMEM((C, EMBED), jnp.float32),
        pltpu.VMEM((C, EMBED), jnp.float32),
        pltpu.SemaphoreType.DMA,
        pltpu.SemaphoreType.DMA,
        pltpu.SemaphoreType.DMA,
        pltpu.SemaphoreType.DMA,
        pltpu.SemaphoreType.DMA,
        pltpu.SemaphoreType.DMA,
        pltpu.SemaphoreType.DMA,
        pltpu.SemaphoreType.DMA,
    ],
)(_sc_body)


@jax.jit
def kernel(toeken_seq, hour_seq, poi_table, hour_table, fc_W, fc_b):
    tok = toeken_seq.reshape(-1).astype(jnp.int32)
    hour = hour_seq.reshape(-1).astype(jnp.int32)
    ht_pad = jnp.zeros((32, EMBED), jnp.float32).at[:25, :HOUR_DIM].set(hour_table)
    w1 = fc_W[:EMBED]
    w2_pad = jnp.zeros((EMBED, EMBED), jnp.float32).at[:HOUR_DIM].set(fc_W[EMBED:])
    poi_c, hp = _precompute(poi_table, w1, ht_pad, w2_pad,
                            fc_b.reshape(1, EMBED), _PE)
    out = _sc_gather(tok, hour, poi_c, hp)
    return out.reshape(B, L, EMBED)


# speedup vs baseline: 3.7262x; 1.0001x over previous
import functools
import numpy as np
import jax
import jax.numpy as jnp
from jax import lax
from jax.experimental import pallas as pl
from jax.experimental.pallas import tpu as pltpu
from jax.experimental.pallas import tpu_sc as plsc

B, L = 4096, 200
POI = 100001
EMBED = 128
HOUR_DIM = 32
HSTRIDE = 32
_RB = 1024
PAD_POI = 100352
_NSTEP = PAD_POI // _RB

NW = 32
TOTAL = B * L
PER_W = TOTAL // NW
C = 128
NCHUNK = PER_W // C


def _sinusoidal_pe(seq_len, d_model):
    pos = np.arange(seq_len, dtype=np.float32)[:, None]
    div = np.exp(np.arange(0, d_model, 2, dtype=np.float32) * (-np.log(10000.0) / d_model))
    pe = np.zeros((seq_len, d_model), dtype=np.float32)
    pe[:, 0::2] = np.sin(pos * div)
    pe[:, 1::2] = np.cos(pos * div)
    return pe


_PE = _sinusoidal_pe(L, EMBED)


def _precompute_body(pt_ref, w1_ref, ht_ref, w2_ref, b_ref, pe_ref, poi_out, hp_out):
    i = pl.program_id(0)
    x = pt_ref[...]
    rid = lax.broadcasted_iota(jnp.int32, (_RB, 1), 0) + i * _RB
    x = jnp.where(rid == 0, 0.0, x)
    poi_out[...] = jnp.dot(x, w1_ref[...], preferred_element_type=jnp.float32)

    @pl.when(i == 0)
    def _():
        h = ht_ref[...]
        hid = lax.broadcasted_iota(jnp.int32, (32, 1), 0)
        h = jnp.where(hid == 0, 0.0, h)
        hc = jnp.dot(h, w2_ref[...], preferred_element_type=jnp.float32) + b_ref[...]
        hp = pe_ref[...][:, None, :] + hc[None, :, :]
        hp_out[...] = hp.reshape(L * HSTRIDE, EMBED)


_precompute = pl.pallas_call(
    _precompute_body,
    grid=(_NSTEP,),
    in_specs=[
        pl.BlockSpec((_RB, EMBED), lambda i: (i, 0)),
        pl.BlockSpec((EMBED, EMBED), lambda i: (0, 0)),
        pl.BlockSpec((32, EMBED), lambda i: (0, 0)),
        pl.BlockSpec((EMBED, EMBED), lambda i: (0, 0)),
        pl.BlockSpec((1, EMBED), lambda i: (0, 0)),
        pl.BlockSpec((L, EMBED), lambda i: (0, 0)),
    ],
    out_specs=[
        pl.BlockSpec((_RB, EMBED), lambda i: (i, 0)),
        pl.BlockSpec((L * HSTRIDE, EMBED), lambda i: (0, 0)),
    ],
    out_shape=[
        jax.ShapeDtypeStruct((PAD_POI, EMBED), jnp.float32),
        jax.ShapeDtypeStruct((L * HSTRIDE, EMBED), jnp.float32),
    ],
)


_NBUF = 4


def _sc_body(tok_hbm, hour_hbm, poi_hbm, hp_hbm, out_hbm,
             tok_v, idx2_v, rows0, rows1, rows2, rows3,
             sg0, sg1, sg2, sg3, so0, so1, so2, so3):
    cid = lax.axis_index("c")
    sid = lax.axis_index("s")
    wid = sid * 2 + cid
    base = wid * PER_W

    rows = (rows0, rows1, rows2, rows3)
    sg = (sg0, sg1, sg2, sg3)
    so = (so0, so1, so2, so3)

    pltpu.sync_copy(tok_hbm.at[pl.ds(base, PER_W)], tok_v)
    pltpu.sync_copy(hour_hbm.at[pl.ds(base, PER_W)], idx2_v)

    l0 = lax.rem(base + lax.iota(jnp.int32, 16), L)

    def mk_idx(k, l):
        for t in range(2):
            off = k * 32 + t * 16
            idx2_v[pl.ds(off, 16)] = l * HSTRIDE + idx2_v[pl.ds(off, 16)]
            ln = l + 16
            l = jnp.where(ln >= L, ln - L, ln)
        return l

    lax.fori_loop(0, PER_W // 32, mk_idx, l0)

    def issue_a(g, j):
        pltpu.async_copy(poi_hbm.at[tok_v.at[pl.ds(g * C, C)]], rows[j], sg[j])

    def issue_b(g, j):
        pltpu.async_copy(hp_hbm.at[idx2_v.at[pl.ds(g * C, C)]], rows[j], sg[j],
                         add=True)

    def wait_g(j):
        pltpu.make_async_copy(poi_hbm.at[tok_v.at[pl.ds(0, C)]], rows[j],
                              sg[j]).wait()

    def issue_out(g, j):
        pltpu.async_copy(rows[j], out_hbm.at[pl.ds(base + g * C, C)], so[j])

    def wait_out(j):
        pltpu.make_async_copy(rows[j], out_hbm.at[pl.ds(base, C)], so[j]).wait()

    c0 = jnp.float32(0.9963463153606634)
    c1 = jnp.float32(-0.3105520803637966)
    c2 = jnp.float32(0.09100609831812505)
    c3 = jnp.float32(-0.016430265990737714)
    c4 = jnp.float32(0.0012641228580639412)

    def compute(j):
        ra = rows[j]

        @plsc.parallel_loop(0, C, unroll=2)
        def _row(r):
            for k in range(EMBED // 16):
                sl = pl.ds(k * 16, 16)
                x = ra[r, sl]
                x = jnp.minimum(jnp.maximum(x, -2.0), 2.0)
                u = x * x
                p = (((c4 * u + c3) * u + c2) * u + c1) * u + c0
                ra[r, sl] = x * p

    NQ = NCHUNK // _NBUF

    issue_a(0, 0)
    issue_a(1, 1)
    wait_g(0)
    issue_b(0, 0)

    def quad(i, _):
        for b in range(_NBUF):
            ja, jb, jc = (b + 2) % _NBUF, (b + 1) % _NBUF, b
            if b >= 2:
                wait_out(ja)

                @pl.when(i < NQ - 1)
                def _(b=b, ja=ja, i_=i):
                    issue_a(i_ * _NBUF + b + 2, ja)
            else:

                @pl.when(i >= 1)
                def _(ja=ja):
                    wait_out(ja)

                issue_a(i * _NBUF + b + 2, ja)
            if b < 3:
                wait_g(jb)
                issue_b(i * _NBUF + b + 1, jb)
            else:

                @pl.when(i < NQ - 1)
                def _(jb=jb, i_=i):
                    wait_g(jb)
                    issue_b(i_ * _NBUF + b + 1, jb)
            wait_g(jc)
            compute(jc)
            issue_out(i * _NBUF + b, jc)
        return 0

    lax.fori_loop(0, NQ, quad, 0)
    wait_out(2)
    wait_out(3)


_sc_gather = functools.partial(
    pl.kernel,
    out_type=jax.ShapeDtypeStruct((TOTAL, EMBED), jnp.float32),
    mesh=plsc.VectorSubcoreMesh(core_axis_name="c", subcore_axis_name="s"),
    scratch_types=[
        pltpu.VMEM((PER_W,), jnp.int32),
        pltpu.VMEM((PER_W,), jnp.int32),
        pltpu.VMEM((C, EMBED), jnp.float32),
        pltpu.VMEM((C, EMBED), jnp.float32),
        pltpu.VMEM((C, EMBED), jnp.float32),
        pltpu.VMEM((C, EMBED), jnp.float32),
        pltpu.SemaphoreType.DMA,
        pltpu.SemaphoreType.DMA,
        pltpu.SemaphoreType.DMA,
        pltpu.SemaphoreType.DMA,
        pltpu.SemaphoreType.DMA,
        pltpu.SemaphoreType.DMA,
        pltpu.SemaphoreType.DMA,
        pltpu.SemaphoreType.DMA,
    ],
)(_sc_body)


@jax.jit
def kernel(toeken_seq, hour_seq, poi_table, hour_table, fc_W, fc_b):
    tok = toeken_seq.reshape(-1).astype(jnp.int32)
    hour = hour_seq.reshape(-1).astype(jnp.int32)
    ht_pad = jnp.zeros((32, EMBED), jnp.float32).at[:25, :HOUR_DIM].set(hour_table)
    w1 = fc_W[:EMBED]
    w2_pad = jnp.zeros((EMBED, EMBED), jnp.float32).at[:HOUR_DIM].set(fc_W[EMBED:])
    poi_c, hp = _precompute(poi_table, w1, ht_pad, w2_pad,
                            fc_b.reshape(1, EMBED), _PE)
    out = _sc_gather(tok, hour, poi_c, hp)
    return out.reshape(B, L, EMBED)
